# Initial kernel scaffold; baseline (speedup 1.0000x reference)
import functools

import jax
import jax.numpy as jnp
from jax import lax
from jax.experimental import pallas as pl
from jax.experimental.pallas import tpu as pltpu
from jax.experimental.pallas import tpu_sc as plsc

D = 128
K = 128
NC = 2
NS = 16
NW = NC * NS
DEGW = 16


def _ceil_to(x, m):
    return ((x + m - 1) // m) * m


@functools.lru_cache(maxsize=None)
def _make_seg2(t1_rows, e1, acc1, t2_rows, e2, acc2):
    npc1 = e1 // (NW * K)
    npc2 = e2 // (NW * K)
    z1 = acc1 // NS
    z2 = acc2 // NS
    mesh = plsc.VectorSubcoreMesh(core_axis_name="c", subcore_axis_name="s")
    out_type = [
        jax.ShapeDtypeStruct((NC, acc1, D), jnp.float32),
        jax.ShapeDtypeStruct((NC, acc1, DEGW), jnp.float32),
        jax.ShapeDtypeStruct((NC, acc2, D), jnp.float32),
        jax.ShapeDtypeStruct((NC, acc2, DEGW), jnp.float32),
    ]
    scratch = [
        pltpu.VMEM_SHARED((max(acc1, acc2), D), jnp.float32),
        pltpu.VMEM_SHARED((max(acc1, acc2), DEGW), jnp.float32),
        pltpu.VMEM((K,), jnp.int32),
        pltpu.VMEM((K,), jnp.int32),
        pltpu.VMEM((K, D), jnp.float32),
        pltpu.VMEM((K, DEGW), jnp.float32),
        pltpu.SemaphoreType.DMA,
    ]

    @functools.partial(pl.kernel, mesh=mesh, out_type=out_type,
                       scratch_types=scratch)
    def seg2(t1, s1, d1, t2, s2, d2, zf, zd, ones_h,
             sum1, deg1, sum2, deg2,
             acc, dacc, isrc, idst, rows, ones_v, sem):
        c = lax.axis_index("c")
        s = lax.axis_index("s")
        wid = s * NC + c
        pltpu.sync_copy(ones_h, ones_v)

        def run_job(table, srcr, dstr, npc, zper, sumo, dego):
            pltpu.sync_copy(zf.at[pl.ds(0, zper)],
                            acc.at[pl.ds(s * zper, zper)])
            pltpu.sync_copy(zd.at[pl.ds(0, zper)],
                            dacc.at[pl.ds(s * zper, zper)])
            plsc.subcore_barrier()

            def body(g, carry):
                base = (wid * npc + g) * K
                pltpu.sync_copy(srcr.at[pl.ds(base, K)], isrc)
                pltpu.sync_copy(dstr.at[pl.ds(base, K)], idst)
                pltpu.async_copy(table.at[isrc], rows, sem).wait()
                pltpu.sync_copy(rows, acc.at[idst], add=True)
                pltpu.sync_copy(ones_v, dacc.at[idst], add=True)
                return carry

            lax.fori_loop(0, npc, body, 0)
            plsc.subcore_barrier()
            pltpu.sync_copy(acc.at[pl.ds(s * zper, zper)],
                            sumo.at[c, pl.ds(s * zper, zper)])
            pltpu.sync_copy(dacc.at[pl.ds(s * zper, zper)],
                            dego.at[c, pl.ds(s * zper, zper)])
            plsc.subcore_barrier()

        run_job(t1, s1, d1, npc1, z1, sum1, deg1)
        run_job(t2, s2, d2, npc2, z2, sum2, deg2)

    return seg2


def _pad_edges(ei, n_dst):
    src, dst = ei[0], ei[1]
    e = src.shape[0]
    epad = _ceil_to(e, NW * K)
    pad = epad - e
    if pad:
        src = jnp.concatenate([src, jnp.zeros((pad,), jnp.int32)])
        dst = jnp.concatenate([dst, jnp.full((pad,), n_dst, jnp.int32)])
    return src, dst, epad


def _mean_linear(sums, degs, W, b, n, leaky):
    blk = 1000
    nb = n // blk

    def body(s_ref, d_ref, w_ref, b_ref, o_ref):
        ss = s_ref[...]
        dd = d_ref[...]
        sm = ss[0] + ss[1]
        d = dd[0, :, 0:1] + dd[1, :, 0:1]
        mean = sm / jnp.maximum(d, 1.0)
        h = jnp.dot(mean, w_ref[...], preferred_element_type=jnp.float32)
        h = h + jnp.where(d > 0, b_ref[...], 0.0)
        if leaky:
            h = jnp.where(h >= 0, h, 0.01 * h)
        o_ref[...] = h

    return pl.pallas_call(
        body,
        grid=(nb,),
        in_specs=[
            pl.BlockSpec((NC, blk, D), lambda i: (0, i, 0)),
            pl.BlockSpec((NC, blk, DEGW), lambda i: (0, i, 0)),
            pl.BlockSpec((D, D), lambda i: (0, 0)),
            pl.BlockSpec((1, D), lambda i: (0, 0)),
        ],
        out_specs=pl.BlockSpec((blk, D), lambda i: (i, 0)),
        out_shape=jax.ShapeDtypeStruct((n, D), jnp.float32),
    )(sums, degs, W, b.reshape(1, D))


def _final_combine(sw, dw, Ww, bw, st, dt, Wt, bt, n):
    blk = 1000
    nb = n // blk

    def body(sw_ref, dw_ref, ww_ref, bw_ref, st_ref, dt_ref, wt_ref, bt_ref,
             o_ref):
        out = None
        for s_ref, d_ref, w_ref, b_ref in (
                (sw_ref, dw_ref, ww_ref, bw_ref),
                (st_ref, dt_ref, wt_ref, bt_ref)):
            ss = s_ref[...]
            dd = d_ref[...]
            sm = ss[0] + ss[1]
            d = dd[0, :, 0:1] + dd[1, :, 0:1]
            mean = sm / jnp.maximum(d, 1.0)
            h = jnp.dot(mean, w_ref[...], preferred_element_type=jnp.float32)
            h = h + jnp.where(d > 0, b_ref[...], 0.0)
            out = h if out is None else out + h
        o_ref[...] = out

    mat = pl.BlockSpec((NC, blk, D), lambda i: (0, i, 0))
    deg = pl.BlockSpec((NC, blk, DEGW), lambda i: (0, i, 0))
    wsp = pl.BlockSpec((D, D), lambda i: (0, 0))
    bsp = pl.BlockSpec((1, D), lambda i: (0, 0))
    return pl.pallas_call(
        body,
        grid=(nb,),
        in_specs=[mat, deg, wsp, bsp, mat, deg, wsp, bsp],
        out_specs=pl.BlockSpec((blk, D), lambda i: (i, 0)),
        out_shape=jax.ShapeDtypeStruct((n, D), jnp.float32),
    )(sw, dw, Ww, bw.reshape(1, D), st, dt, Wt, bt.reshape(1, D))


def kernel(embeds, params, edges):
    paper = embeds["paper"]
    n_author, n_paper, n_field = 10000, 10000, 5000

    W_wb, b_wb = params["layer0"]["paper,written_by,author"]
    W_ht, b_ht = params["layer0"]["paper,has_topic,field"]
    W_w, b_w = params["layer1"]["author,writes,paper"]
    W_t, b_t = params["layer1"]["field,topic_of,paper"]

    s_wb, d_wb, e_wb = _pad_edges(edges["paper,written_by,author"], n_author)
    s_ht, d_ht, e_ht = _pad_edges(edges["paper,has_topic,field"], n_field)
    s_w, d_w, e_w = _pad_edges(edges["author,writes,paper"], n_paper)
    s_t, d_t, e_t = _pad_edges(edges["field,topic_of,paper"], n_paper)

    acc_a = _ceil_to(n_author + 8, NS * 8)
    acc_f = _ceil_to(n_field + 8, NS * 8)
    acc_p = _ceil_to(n_paper + 8, NS * 8)
    zmax = max(acc_a, acc_f, acc_p) // NS
    zf = jnp.zeros((zmax, D), jnp.float32)
    zd = jnp.zeros((zmax, DEGW), jnp.float32)
    ones = jnp.ones((K, DEGW), jnp.float32)

    seg_l0 = _make_seg2(n_paper, e_wb, acc_a, n_paper, e_ht, acc_f)
    sum_a, deg_a, sum_f, deg_f = seg_l0(paper, s_wb, d_wb, paper, s_ht, d_ht,
                                        zf, zd, ones)
    h_a = _mean_linear(sum_a[:, :n_author], deg_a[:, :n_author],
                       W_wb, b_wb, n_author, leaky=True)
    h_f = _mean_linear(sum_f[:, :n_field], deg_f[:, :n_field],
                       W_ht, b_ht, n_field, leaky=True)

    seg_l1 = _make_seg2(n_author, e_w, acc_p, n_field, e_t, acc_p)
    sum_w, deg_w, sum_t, deg_t = seg_l1(h_a, s_w, d_w, h_f, s_t, d_t,
                                        zf, zd, ones)
    return _final_combine(sum_w[:, :n_paper], deg_w[:, :n_paper], W_w, b_w,
                          sum_t[:, :n_paper], deg_t[:, :n_paper], W_t, b_t,
                          n_paper)


# speedup vs baseline: 2.3770x; 2.3770x over previous
import functools

import jax
import jax.numpy as jnp
from jax import lax
from jax.experimental import pallas as pl
from jax.experimental.pallas import tpu as pltpu
from jax.experimental.pallas import tpu_sc as plsc

D = 128
K = 128
NC = 2
NS = 16
NW = NC * NS
DEGW = 128


def _ceil_to(x, m):
    return ((x + m - 1) // m) * m


@functools.lru_cache(maxsize=None)
def _make_deg(jobs):
    accmax = max(a for _, a in jobs)
    mesh = plsc.VectorSubcoreMesh(core_axis_name="c", subcore_axis_name="s")
    out_type = [jax.ShapeDtypeStruct((NC, a, DEGW), jnp.float32)
                for _, a in jobs]
    scratch = [
        pltpu.VMEM_SHARED((accmax, DEGW), jnp.float32),
        pltpu.VMEM((K,), jnp.int32),
        pltpu.VMEM((K, DEGW), jnp.float32),
    ]

    @functools.partial(pl.kernel, mesh=mesh, out_type=out_type,
                       scratch_types=scratch)
    def deg_kernel(*args):
        nj = len(jobs)
        dsts = args[:nj]
        zd, ones_h = args[nj], args[nj + 1]
        outs = args[nj + 2:2 * nj + 2]
        dacc, idst, dbuf = args[2 * nj + 2:]
        c = lax.axis_index("c")
        s = lax.axis_index("s")
        wid = s * NC + c

        for (e, acc), dstr, dego in zip(jobs, dsts, outs):
            npc = e // (NW * K)
            z = acc // NS
            nzc = z // K
            pltpu.sync_copy(zd, dbuf)

            def zbody(i, carry):
                pltpu.sync_copy(dbuf, dacc.at[pl.ds(s * z + i * K, K)])
                return carry

            lax.fori_loop(0, nzc, zbody, 0)
            plsc.subcore_barrier()
            pltpu.sync_copy(ones_h, dbuf)

            def body(g, carry):
                base = (wid * npc + g) * K
                pltpu.sync_copy(dstr.at[pl.ds(base, K)], idst)
                pltpu.sync_copy(dbuf, dacc.at[idst], add=True)
                return carry

            lax.fori_loop(0, npc, body, 0)
            plsc.subcore_barrier()

            def obody(i, carry):
                r0 = s * z + i * K
                pltpu.sync_copy(dacc.at[pl.ds(r0, K)], dbuf)
                pltpu.sync_copy(dbuf, dego.at[c, pl.ds(r0, K)])
                return carry

            lax.fori_loop(0, nzc, obody, 0)
            plsc.subcore_barrier()

    return deg_kernel


@functools.lru_cache(maxsize=None)
def _make_feat2(e1, acc1, e2, acc2):
    npcs = (e1 // (NW * K), e2 // (NW * K))
    zs = (acc1 // NS, acc2 // NS)
    mesh = plsc.VectorSubcoreMesh(core_axis_name="c", subcore_axis_name="s")
    out_type = [
        jax.ShapeDtypeStruct((NC, acc1, D), jnp.float32),
        jax.ShapeDtypeStruct((NC, acc2, D), jnp.float32),
    ]
    scratch = [
        pltpu.VMEM_SHARED((max(acc1, acc2), D), jnp.float32),
        pltpu.VMEM((K,), jnp.int32),
        pltpu.VMEM((K,), jnp.int32),
        pltpu.VMEM((K, D), jnp.float32),
        pltpu.SemaphoreType.DMA,
    ]

    @functools.partial(pl.kernel, mesh=mesh, out_type=out_type,
                       scratch_types=scratch)
    def feat2(t1, s1, d1, t2, s2, d2, zf,
              sum1, sum2,
              acc, isrc, idst, rows, sem):
        c = lax.axis_index("c")
        s = lax.axis_index("s")
        wid = s * NC + c

        for (table, srcr, dstr, npc, z, sumo) in (
                (t1, s1, d1, npcs[0], zs[0], sum1),
                (t2, s2, d2, npcs[1], zs[1], sum2)):
            nzc = z // K
            pltpu.sync_copy(zf, rows)

            def zbody(i, carry):
                pltpu.sync_copy(rows, acc.at[pl.ds(s * z + i * K, K)])
                return carry

            lax.fori_loop(0, nzc, zbody, 0)
            plsc.subcore_barrier()

            def body(g, carry):
                base = (wid * npc + g) * K
                pltpu.sync_copy(srcr.at[pl.ds(base, K)], isrc)
                pltpu.sync_copy(dstr.at[pl.ds(base, K)], idst)
                pltpu.async_copy(table.at[isrc], rows, sem).wait()
                pltpu.sync_copy(rows, acc.at[idst], add=True)
                return carry

            lax.fori_loop(0, npc, body, 0)
            plsc.subcore_barrier()

            def obody(i, carry):
                r0 = s * z + i * K
                pltpu.sync_copy(acc.at[pl.ds(r0, K)], rows)
                pltpu.sync_copy(rows, sumo.at[c, pl.ds(r0, K)])
                return carry

            lax.fori_loop(0, nzc, obody, 0)
            plsc.subcore_barrier()

    return feat2


def _pad_edges(ei, n_dst):
    src, dst = ei[0], ei[1]
    e = src.shape[0]
    epad = _ceil_to(e, NW * K)
    pad = epad - e
    if pad:
        src = jnp.concatenate([src, jnp.zeros((pad,), jnp.int32)])
        dst = jnp.concatenate([dst, jnp.full((pad,), n_dst, jnp.int32)])
    return src, dst, epad


def _mean_linear(sums, degs, W, b, n, leaky):
    blk = 1000
    nb = n // blk

    def body(s_ref, d_ref, w_ref, b_ref, o_ref):
        ss = s_ref[...]
        dd = d_ref[...]
        sm = ss[0] + ss[1]
        d = dd[0, :, 0:1] + dd[1, :, 0:1]
        mean = sm / jnp.maximum(d, 1.0)
        h = jnp.dot(mean, w_ref[...], preferred_element_type=jnp.float32)
        h = h + jnp.where(d > 0, b_ref[...], 0.0)
        if leaky:
            h = jnp.where(h >= 0, h, 0.01 * h)
        o_ref[...] = h

    return pl.pallas_call(
        body,
        grid=(nb,),
        in_specs=[
            pl.BlockSpec((NC, blk, D), lambda i: (0, i, 0)),
            pl.BlockSpec((NC, blk, DEGW), lambda i: (0, i, 0)),
            pl.BlockSpec((D, D), lambda i: (0, 0)),
            pl.BlockSpec((1, D), lambda i: (0, 0)),
        ],
        out_specs=pl.BlockSpec((blk, D), lambda i: (i, 0)),
        out_shape=jax.ShapeDtypeStruct((n, D), jnp.float32),
    )(sums, degs, W, b.reshape(1, D))


def _final_combine(sw, dw, Ww, bw, st, dt, Wt, bt, n):
    blk = 1000
    nb = n // blk

    def body(sw_ref, dw_ref, ww_ref, bw_ref, st_ref, dt_ref, wt_ref, bt_ref,
             o_ref):
        out = None
        for s_ref, d_ref, w_ref, b_ref in (
                (sw_ref, dw_ref, ww_ref, bw_ref),
                (st_ref, dt_ref, wt_ref, bt_ref)):
            ss = s_ref[...]
            dd = d_ref[...]
            sm = ss[0] + ss[1]
            d = dd[0, :, 0:1] + dd[1, :, 0:1]
            mean = sm / jnp.maximum(d, 1.0)
            h = jnp.dot(mean, w_ref[...], preferred_element_type=jnp.float32)
            h = h + jnp.where(d > 0, b_ref[...], 0.0)
            out = h if out is None else out + h
        o_ref[...] = out

    mat = pl.BlockSpec((NC, blk, D), lambda i: (0, i, 0))
    deg = pl.BlockSpec((NC, blk, DEGW), lambda i: (0, i, 0))
    wsp = pl.BlockSpec((D, D), lambda i: (0, 0))
    bsp = pl.BlockSpec((1, D), lambda i: (0, 0))
    return pl.pallas_call(
        body,
        grid=(nb,),
        in_specs=[mat, deg, wsp, bsp, mat, deg, wsp, bsp],
        out_specs=pl.BlockSpec((blk, D), lambda i: (i, 0)),
        out_shape=jax.ShapeDtypeStruct((n, D), jnp.float32),
    )(sw, dw, Ww, bw.reshape(1, D), st, dt, Wt, bt.reshape(1, D))


def kernel(embeds, params, edges):
    paper = embeds["paper"]
    n_author, n_paper, n_field = 10000, 10000, 5000

    W_wb, b_wb = params["layer0"]["paper,written_by,author"]
    W_ht, b_ht = params["layer0"]["paper,has_topic,field"]
    W_w, b_w = params["layer1"]["author,writes,paper"]
    W_t, b_t = params["layer1"]["field,topic_of,paper"]

    s_wb, d_wb, e_wb = _pad_edges(edges["paper,written_by,author"], n_author)
    s_ht, d_ht, e_ht = _pad_edges(edges["paper,has_topic,field"], n_field)
    s_w, d_w, e_w = _pad_edges(edges["author,writes,paper"], n_paper)
    s_t, d_t, e_t = _pad_edges(edges["field,topic_of,paper"], n_paper)

    acc_a = _ceil_to(n_author + 8, NS * K)
    acc_f = _ceil_to(n_field + 8, NS * K)
    acc_p = _ceil_to(n_paper + 8, NS * K)
    zf = jnp.zeros((K, D), jnp.float32)
    zd = jnp.zeros((K, DEGW), jnp.float32)
    ones = jnp.ones((K, DEGW), jnp.float32)

    deg_k = _make_deg(((e_wb, acc_a), (e_ht, acc_f), (e_w, acc_p),
                       (e_t, acc_p)))
    deg_a, deg_f, deg_w, deg_t = deg_k(d_wb, d_ht, d_w, d_t, zd, ones)

    sum_a, sum_f = _make_feat2(e_wb, acc_a, e_ht, acc_f)(
        paper, s_wb, d_wb, paper, s_ht, d_ht, zf)
    h_a = _mean_linear(sum_a[:, :n_author], deg_a[:, :n_author],
                       W_wb, b_wb, n_author, leaky=True)
    h_f = _mean_linear(sum_f[:, :n_field], deg_f[:, :n_field],
                       W_ht, b_ht, n_field, leaky=True)

    sum_w, sum_t = _make_feat2(e_w, acc_p, e_t, acc_p)(
        h_a, s_w, d_w, h_f, s_t, d_t, zf)
    return _final_combine(sum_w[:, :n_paper], deg_w[:, :n_paper], W_w, b_w,
                          sum_t[:, :n_paper], deg_t[:, :n_paper], W_t, b_t,
                          n_paper)


# speedup vs baseline: 2.6828x; 1.1287x over previous
import functools

import jax
import jax.numpy as jnp
from jax import lax
from jax.experimental import pallas as pl
from jax.experimental.pallas import tpu as pltpu
from jax.experimental.pallas import tpu_sc as plsc

D = 128
K = 128
NC = 2
NS = 16
NW = NC * NS
DEGW = 128
NB = 4


def _ceil_to(x, m):
    return ((x + m - 1) // m) * m


@functools.lru_cache(maxsize=None)
def _make_deg(jobs):
    accmax = max(a for _, a in jobs)
    mesh = plsc.VectorSubcoreMesh(core_axis_name="c", subcore_axis_name="s")
    out_type = [jax.ShapeDtypeStruct((NC, a, DEGW), jnp.float32)
                for _, a in jobs]
    scratch = [
        pltpu.VMEM_SHARED((accmax, DEGW), jnp.float32),
        pltpu.VMEM((NB, K), jnp.int32),
        pltpu.VMEM((K, DEGW), jnp.float32),
        pltpu.SemaphoreType.DMA,
    ]

    @functools.partial(pl.kernel, mesh=mesh, out_type=out_type,
                       scratch_types=scratch)
    def deg_kernel(*args):
        nj = len(jobs)
        dsts = args[:nj]
        zd, ones_h = args[nj], args[nj + 1]
        outs = args[nj + 2:2 * nj + 2]
        dacc, dblk, dbuf, ssc = args[2 * nj + 2:]
        c = lax.axis_index("c")
        s = lax.axis_index("s")
        wid = s * NC + c

        for (e, acc), dstr, dego in zip(jobs, dsts, outs):
            npc = e // (NW * K)
            z = acc // NS
            nzc = z // K
            pltpu.sync_copy(zd, dbuf)

            def zbody(i, carry):
                pltpu.sync_copy(dbuf, dacc.at[pl.ds(s * z + i * K, K)])
                return carry

            lax.fori_loop(0, nzc, zbody, 0)
            plsc.subcore_barrier()
            pltpu.sync_copy(ones_h, dbuf)
            crow0 = wid * npc

            def body(t, carry):
                pltpu.sync_copy(dstr.at[pl.ds(crow0 + t * NB, NB)], dblk)
                descs = [pltpu.async_copy(dbuf, dacc.at[dblk.at[u]], ssc,
                                          add=True) for u in range(NB)]
                for dsc in descs:
                    dsc.wait()
                return carry

            lax.fori_loop(0, npc // NB, body, 0)
            plsc.subcore_barrier()

            def obody(i, carry):
                r0 = s * z + i * K
                pltpu.sync_copy(dacc.at[pl.ds(r0, K)], dbuf)
                pltpu.sync_copy(dbuf, dego.at[c, pl.ds(r0, K)])
                return carry

            lax.fori_loop(0, nzc, obody, 0)
            plsc.subcore_barrier()

    return deg_kernel


@functools.lru_cache(maxsize=None)
def _make_feat2(e1, acc1, e2, acc2):
    npcs = (e1 // (NW * K), e2 // (NW * K))
    zs = (acc1 // NS, acc2 // NS)
    mesh = plsc.VectorSubcoreMesh(core_axis_name="c", subcore_axis_name="s")
    out_type = [
        jax.ShapeDtypeStruct((NC, acc1, D), jnp.float32),
        jax.ShapeDtypeStruct((NC, acc2, D), jnp.float32),
    ]
    scratch = [
        pltpu.VMEM_SHARED((max(acc1, acc2), D), jnp.float32),
        pltpu.VMEM((NB, K), jnp.int32),
        pltpu.VMEM((NB, K), jnp.int32),
        pltpu.VMEM((K, D), jnp.float32),
        pltpu.VMEM((K, D), jnp.float32),
        pltpu.SemaphoreType.DMA,
        pltpu.SemaphoreType.DMA,
        pltpu.SemaphoreType.DMA,
        pltpu.SemaphoreType.DMA,
    ]

    @functools.partial(pl.kernel, mesh=mesh, out_type=out_type,
                       scratch_types=scratch)
    def feat2(t1, s1, d1, t2, s2, d2, zf,
              sum1, sum2,
              acc, sblk, dblk, rows0, rows1, sg0, sg1, ss0, ss1):
        c = lax.axis_index("c")
        s = lax.axis_index("s")
        wid = s * NC + c

        for (table, srcr, dstr, npc, z, sumo) in (
                (t1, s1, d1, npcs[0], zs[0], sum1),
                (t2, s2, d2, npcs[1], zs[1], sum2)):
            nzc = z // K
            pltpu.sync_copy(zf, rows0)

            def zbody(i, carry):
                pltpu.sync_copy(rows0, acc.at[pl.ds(s * z + i * K, K)])
                return carry

            lax.fori_loop(0, nzc, zbody, 0)
            plsc.subcore_barrier()
            crow0 = wid * npc

            def body(t, carry):
                crow = crow0 + t * NB
                i0 = pltpu.async_copy(srcr.at[pl.ds(crow, NB)], sblk, sg0)
                i1 = pltpu.async_copy(dstr.at[pl.ds(crow, NB)], dblk, sg1)
                i0.wait()
                i1.wait()
                cprev = None
                for h in range(NB // 2):
                    if cprev is not None:
                        cprev[0].wait()
                    g0 = pltpu.async_copy(table.at[sblk.at[2 * h]],
                                          rows0, sg0)
                    if cprev is not None:
                        cprev[1].wait()
                    g1 = pltpu.async_copy(table.at[sblk.at[2 * h + 1]],
                                          rows1, sg1)
                    g0.wait()
                    c0 = pltpu.async_copy(rows0, acc.at[dblk.at[2 * h]],
                                          ss0, add=True)
                    g1.wait()
                    c1 = pltpu.async_copy(rows1, acc.at[dblk.at[2 * h + 1]],
                                          ss1, add=True)
                    cprev = (c0, c1)
                cprev[0].wait()
                cprev[1].wait()
                return carry

            lax.fori_loop(0, npc // NB, body, 0)
            plsc.subcore_barrier()

            def obody(i, carry):
                r0 = s * z + i * K
                pltpu.sync_copy(acc.at[pl.ds(r0, K)], rows0)
                pltpu.sync_copy(rows0, sumo.at[c, pl.ds(r0, K)])
                return carry

            lax.fori_loop(0, nzc, obody, 0)
            plsc.subcore_barrier()

    return feat2


def _pad_edges(ei, n_dst):
    src, dst = ei[0], ei[1]
    e = src.shape[0]
    epad = _ceil_to(e, NW * K * NB)
    pad = epad - e
    if pad:
        src = jnp.concatenate([src, jnp.zeros((pad,), jnp.int32)])
        dst = jnp.concatenate([dst, jnp.full((pad,), n_dst, jnp.int32)])
    return src.reshape(epad // K, K), dst.reshape(epad // K, K), epad


def _mean_linear(sums, degs, W, b, n, leaky):
    blk = 1000
    nb = n // blk

    def body(s_ref, d_ref, w_ref, b_ref, o_ref):
        ss = s_ref[...]
        dd = d_ref[...]
        sm = ss[0] + ss[1]
        d = dd[0, :, 0:1] + dd[1, :, 0:1]
        mean = sm / jnp.maximum(d, 1.0)
        h = jnp.dot(mean, w_ref[...], preferred_element_type=jnp.float32)
        h = h + jnp.where(d > 0, b_ref[...], 0.0)
        if leaky:
            h = jnp.where(h >= 0, h, 0.01 * h)
        o_ref[...] = h

    return pl.pallas_call(
        body,
        grid=(nb,),
        in_specs=[
            pl.BlockSpec((NC, blk, D), lambda i: (0, i, 0)),
            pl.BlockSpec((NC, blk, DEGW), lambda i: (0, i, 0)),
            pl.BlockSpec((D, D), lambda i: (0, 0)),
            pl.BlockSpec((1, D), lambda i: (0, 0)),
        ],
        out_specs=pl.BlockSpec((blk, D), lambda i: (i, 0)),
        out_shape=jax.ShapeDtypeStruct((n, D), jnp.float32),
    )(sums, degs, W, b.reshape(1, D))


def _final_combine(sw, dw, Ww, bw, st, dt, Wt, bt, n):
    blk = 1000
    nb = n // blk

    def body(sw_ref, dw_ref, ww_ref, bw_ref, st_ref, dt_ref, wt_ref, bt_ref,
             o_ref):
        out = None
        for s_ref, d_ref, w_ref, b_ref in (
                (sw_ref, dw_ref, ww_ref, bw_ref),
                (st_ref, dt_ref, wt_ref, bt_ref)):
            ss = s_ref[...]
            dd = d_ref[...]
            sm = ss[0] + ss[1]
            d = dd[0, :, 0:1] + dd[1, :, 0:1]
            mean = sm / jnp.maximum(d, 1.0)
            h = jnp.dot(mean, w_ref[...], preferred_element_type=jnp.float32)
            h = h + jnp.where(d > 0, b_ref[...], 0.0)
            out = h if out is None else out + h
        o_ref[...] = out

    mat = pl.BlockSpec((NC, blk, D), lambda i: (0, i, 0))
    deg = pl.BlockSpec((NC, blk, DEGW), lambda i: (0, i, 0))
    wsp = pl.BlockSpec((D, D), lambda i: (0, 0))
    bsp = pl.BlockSpec((1, D), lambda i: (0, 0))
    return pl.pallas_call(
        body,
        grid=(nb,),
        in_specs=[mat, deg, wsp, bsp, mat, deg, wsp, bsp],
        out_specs=pl.BlockSpec((blk, D), lambda i: (i, 0)),
        out_shape=jax.ShapeDtypeStruct((n, D), jnp.float32),
    )(sw, dw, Ww, bw.reshape(1, D), st, dt, Wt, bt.reshape(1, D))


def kernel(embeds, params, edges):
    paper = embeds["paper"]
    n_author, n_paper, n_field = 10000, 10000, 5000

    W_wb, b_wb = params["layer0"]["paper,written_by,author"]
    W_ht, b_ht = params["layer0"]["paper,has_topic,field"]
    W_w, b_w = params["layer1"]["author,writes,paper"]
    W_t, b_t = params["layer1"]["field,topic_of,paper"]

    s_wb, d_wb, e_wb = _pad_edges(edges["paper,written_by,author"], n_author)
    s_ht, d_ht, e_ht = _pad_edges(edges["paper,has_topic,field"], n_field)
    s_w, d_w, e_w = _pad_edges(edges["author,writes,paper"], n_paper)
    s_t, d_t, e_t = _pad_edges(edges["field,topic_of,paper"], n_paper)

    acc_a = _ceil_to(n_author + 8, NS * K)
    acc_f = _ceil_to(n_field + 8, NS * K)
    acc_p = _ceil_to(n_paper + 8, NS * K)
    zf = jnp.zeros((K, D), jnp.float32)
    zd = jnp.zeros((K, DEGW), jnp.float32)
    ones = jnp.ones((K, DEGW), jnp.float32)

    deg_k = _make_deg(((e_wb, acc_a), (e_ht, acc_f), (e_w, acc_p),
                       (e_t, acc_p)))
    deg_a, deg_f, deg_w, deg_t = deg_k(d_wb, d_ht, d_w, d_t, zd, ones)

    sum_a, sum_f = _make_feat2(e_wb, acc_a, e_ht, acc_f)(
        paper, s_wb, d_wb, paper, s_ht, d_ht, zf)
    h_a = _mean_linear(sum_a[:, :n_author], deg_a[:, :n_author],
                       W_wb, b_wb, n_author, leaky=True)
    h_f = _mean_linear(sum_f[:, :n_field], deg_f[:, :n_field],
                       W_ht, b_ht, n_field, leaky=True)

    sum_w, sum_t = _make_feat2(e_w, acc_p, e_t, acc_p)(
        h_a, s_w, d_w, h_f, s_t, d_t, zf)
    return _final_combine(sum_w[:, :n_paper], deg_w[:, :n_paper], W_w, b_w,
                          sum_t[:, :n_paper], deg_t[:, :n_paper], W_t, b_t,
                          n_paper)


# speedup vs baseline: 2.8385x; 1.0581x over previous
import functools

import jax
import jax.numpy as jnp
from jax import lax
from jax.experimental import pallas as pl
from jax.experimental.pallas import tpu as pltpu
from jax.experimental.pallas import tpu_sc as plsc

D = 128
K = 128
NC = 2
NS = 16
NW = NC * NS
DEGW = 128
NB = 4


def _ceil_to(x, m):
    return ((x + m - 1) // m) * m


@functools.lru_cache(maxsize=None)
def _make_deg(jobs):
    accmax = max(a for _, a in jobs)
    mesh = plsc.VectorSubcoreMesh(core_axis_name="c", subcore_axis_name="s")
    out_type = [jax.ShapeDtypeStruct((NC, a, DEGW), jnp.float32)
                for _, a in jobs]
    scratch = [
        pltpu.VMEM_SHARED((accmax, DEGW), jnp.float32),
        pltpu.VMEM((NB, K), jnp.int32),
        pltpu.VMEM((K, DEGW), jnp.float32),
        pltpu.SemaphoreType.DMA,
    ]

    @functools.partial(pl.kernel, mesh=mesh, out_type=out_type,
                       scratch_types=scratch)
    def deg_kernel(*args):
        nj = len(jobs)
        dsts = args[:nj]
        zd, ones_h = args[nj], args[nj + 1]
        outs = args[nj + 2:2 * nj + 2]
        dacc, dblk, dbuf, ssc = args[2 * nj + 2:]
        c = lax.axis_index("c")
        s = lax.axis_index("s")
        wid = s * NC + c

        for (e, acc), dstr, dego in zip(jobs, dsts, outs):
            npc = e // (NW * K)
            z = acc // NS
            nzc = z // K
            pltpu.sync_copy(zd, dbuf)

            def zbody(i, carry):
                pltpu.sync_copy(dbuf, dacc.at[pl.ds(s * z + i * K, K)])
                return carry

            lax.fori_loop(0, nzc, zbody, 0)
            plsc.subcore_barrier()
            pltpu.sync_copy(ones_h, dbuf)
            crow0 = wid * npc

            def body(t, carry):
                pltpu.sync_copy(dstr.at[pl.ds(crow0 + t * NB, NB)], dblk)
                descs = [pltpu.async_copy(dbuf, dacc.at[dblk.at[u]], ssc,
                                          add=True) for u in range(NB)]
                for dsc in descs:
                    dsc.wait()
                return carry

            lax.fori_loop(0, npc // NB, body, 0)
            plsc.subcore_barrier()

            def obody(i, carry):
                r0 = s * z + i * K
                pltpu.sync_copy(dacc.at[pl.ds(r0, K)], dbuf)
                pltpu.sync_copy(dbuf, dego.at[c, pl.ds(r0, K)])
                return carry

            lax.fori_loop(0, nzc, obody, 0)
            plsc.subcore_barrier()

    return deg_kernel


@functools.lru_cache(maxsize=None)
def _make_feat2(e1, acc1, e2, acc2):
    npcs = (e1 // (NW * K), e2 // (NW * K))
    zs = (acc1 // NS, acc2 // NS)
    mesh = plsc.VectorSubcoreMesh(core_axis_name="c", subcore_axis_name="s")
    out_type = [
        jax.ShapeDtypeStruct((NC, acc1, D), jnp.float32),
        jax.ShapeDtypeStruct((NC, acc2, D), jnp.float32),
    ]
    scratch = [
        pltpu.VMEM_SHARED((max(acc1, acc2), D), jnp.float32),
        pltpu.VMEM((NB, K), jnp.int32),
        pltpu.VMEM((NB, K), jnp.int32),
        pltpu.VMEM((K, D), jnp.float32),
        pltpu.VMEM((K, D), jnp.float32),
        pltpu.SemaphoreType.DMA,
        pltpu.SemaphoreType.DMA,
        pltpu.SemaphoreType.DMA,
        pltpu.SemaphoreType.DMA,
    ]

    @functools.partial(pl.kernel, mesh=mesh, out_type=out_type,
                       scratch_types=scratch)
    def feat2(t1, s1, d1, t2, s2, d2, zf,
              sum1, sum2,
              acc, sblk, dblk, rows0, rows1, sg0, sg1, ss0, ss1):
        c = lax.axis_index("c")
        s = lax.axis_index("s")
        wid = s * NC + c

        for (table, srcr, dstr, npc, z, sumo) in (
                (t1, s1, d1, npcs[0], zs[0], sum1),
                (t2, s2, d2, npcs[1], zs[1], sum2)):
            nzc = z // K
            pltpu.sync_copy(zf, rows0)

            def zbody(i, carry):
                pltpu.sync_copy(rows0, acc.at[pl.ds(s * z + i * K, K)])
                return carry

            lax.fori_loop(0, nzc, zbody, 0)
            plsc.subcore_barrier()
            crow0 = wid * npc

            def body(t, carry):
                crow = crow0 + t * NB
                i0 = pltpu.async_copy(srcr.at[pl.ds(crow, NB)], sblk, sg0)
                i1 = pltpu.async_copy(dstr.at[pl.ds(crow, NB)], dblk, sg1)
                i0.wait()
                i1.wait()
                cprev = None
                for h in range(NB // 2):
                    g0 = pltpu.async_copy(table.at[sblk.at[2 * h]],
                                          rows0, sg0)
                    g1 = pltpu.async_copy(table.at[sblk.at[2 * h + 1]],
                                          rows1, sg1)
                    g0.wait()
                    g1.wait()
                    cprev = None
                return carry

            lax.fori_loop(0, npc // NB, body, 0)
            plsc.subcore_barrier()

            def obody(i, carry):
                r0 = s * z + i * K
                pltpu.sync_copy(acc.at[pl.ds(r0, K)], rows0)
                pltpu.sync_copy(rows0, sumo.at[c, pl.ds(r0, K)])
                return carry

            lax.fori_loop(0, nzc, obody, 0)
            plsc.subcore_barrier()

    return feat2


def _pad_edges(ei, n_dst):
    src, dst = ei[0], ei[1]
    e = src.shape[0]
    epad = _ceil_to(e, NW * K * NB)
    pad = epad - e
    if pad:
        src = jnp.concatenate([src, jnp.zeros((pad,), jnp.int32)])
        dst = jnp.concatenate([dst, jnp.full((pad,), n_dst, jnp.int32)])
    return src.reshape(epad // K, K), dst.reshape(epad // K, K), epad


def _mean_linear(sums, degs, W, b, n, leaky):
    blk = 1000
    nb = n // blk

    def body(s_ref, d_ref, w_ref, b_ref, o_ref):
        ss = s_ref[...]
        dd = d_ref[...]
        sm = ss[0] + ss[1]
        d = dd[0, :, 0:1] + dd[1, :, 0:1]
        mean = sm / jnp.maximum(d, 1.0)
        h = jnp.dot(mean, w_ref[...], preferred_element_type=jnp.float32)
        h = h + jnp.where(d > 0, b_ref[...], 0.0)
        if leaky:
            h = jnp.where(h >= 0, h, 0.01 * h)
        o_ref[...] = h

    return pl.pallas_call(
        body,
        grid=(nb,),
        in_specs=[
            pl.BlockSpec((NC, blk, D), lambda i: (0, i, 0)),
            pl.BlockSpec((NC, blk, DEGW), lambda i: (0, i, 0)),
            pl.BlockSpec((D, D), lambda i: (0, 0)),
            pl.BlockSpec((1, D), lambda i: (0, 0)),
        ],
        out_specs=pl.BlockSpec((blk, D), lambda i: (i, 0)),
        out_shape=jax.ShapeDtypeStruct((n, D), jnp.float32),
    )(sums, degs, W, b.reshape(1, D))


def _final_combine(sw, dw, Ww, bw, st, dt, Wt, bt, n):
    blk = 1000
    nb = n // blk

    def body(sw_ref, dw_ref, ww_ref, bw_ref, st_ref, dt_ref, wt_ref, bt_ref,
             o_ref):
        out = None
        for s_ref, d_ref, w_ref, b_ref in (
                (sw_ref, dw_ref, ww_ref, bw_ref),
                (st_ref, dt_ref, wt_ref, bt_ref)):
            ss = s_ref[...]
            dd = d_ref[...]
            sm = ss[0] + ss[1]
            d = dd[0, :, 0:1] + dd[1, :, 0:1]
            mean = sm / jnp.maximum(d, 1.0)
            h = jnp.dot(mean, w_ref[...], preferred_element_type=jnp.float32)
            h = h + jnp.where(d > 0, b_ref[...], 0.0)
            out = h if out is None else out + h
        o_ref[...] = out

    mat = pl.BlockSpec((NC, blk, D), lambda i: (0, i, 0))
    deg = pl.BlockSpec((NC, blk, DEGW), lambda i: (0, i, 0))
    wsp = pl.BlockSpec((D, D), lambda i: (0, 0))
    bsp = pl.BlockSpec((1, D), lambda i: (0, 0))
    return pl.pallas_call(
        body,
        grid=(nb,),
        in_specs=[mat, deg, wsp, bsp, mat, deg, wsp, bsp],
        out_specs=pl.BlockSpec((blk, D), lambda i: (i, 0)),
        out_shape=jax.ShapeDtypeStruct((n, D), jnp.float32),
    )(sw, dw, Ww, bw.reshape(1, D), st, dt, Wt, bt.reshape(1, D))


def kernel(embeds, params, edges):
    paper = embeds["paper"]
    n_author, n_paper, n_field = 10000, 10000, 5000

    W_wb, b_wb = params["layer0"]["paper,written_by,author"]
    W_ht, b_ht = params["layer0"]["paper,has_topic,field"]
    W_w, b_w = params["layer1"]["author,writes,paper"]
    W_t, b_t = params["layer1"]["field,topic_of,paper"]

    s_wb, d_wb, e_wb = _pad_edges(edges["paper,written_by,author"], n_author)
    s_ht, d_ht, e_ht = _pad_edges(edges["paper,has_topic,field"], n_field)
    s_w, d_w, e_w = _pad_edges(edges["author,writes,paper"], n_paper)
    s_t, d_t, e_t = _pad_edges(edges["field,topic_of,paper"], n_paper)

    acc_a = _ceil_to(n_author + 8, NS * K)
    acc_f = _ceil_to(n_field + 8, NS * K)
    acc_p = _ceil_to(n_paper + 8, NS * K)
    zf = jnp.zeros((K, D), jnp.float32)
    zd = jnp.zeros((K, DEGW), jnp.float32)
    ones = jnp.ones((K, DEGW), jnp.float32)

    deg_k = _make_deg(((e_wb, acc_a), (e_ht, acc_f), (e_w, acc_p),
                       (e_t, acc_p)))
    deg_a, deg_f, deg_w, deg_t = deg_k(d_wb, d_ht, d_w, d_t, zd, ones)

    sum_a, sum_f = _make_feat2(e_wb, acc_a, e_ht, acc_f)(
        paper, s_wb, d_wb, paper, s_ht, d_ht, zf)
    h_a = _mean_linear(sum_a[:, :n_author], deg_a[:, :n_author],
                       W_wb, b_wb, n_author, leaky=True)
    h_f = _mean_linear(sum_f[:, :n_field], deg_f[:, :n_field],
                       W_ht, b_ht, n_field, leaky=True)

    sum_w, sum_t = _make_feat2(e_w, acc_p, e_t, acc_p)(
        h_a, s_w, d_w, h_f, s_t, d_t, zf)
    return _final_combine(sum_w[:, :n_paper], deg_w[:, :n_paper], W_w, b_w,
                          sum_t[:, :n_paper], deg_t[:, :n_paper], W_t, b_t,
                          n_paper)


# speedup vs baseline: 3.0060x; 1.0590x over previous
import functools

import jax
import jax.numpy as jnp
from jax import lax
from jax.experimental import pallas as pl
from jax.experimental.pallas import tpu as pltpu
from jax.experimental.pallas import tpu_sc as plsc

D = 128
K = 128
NC = 2
NS = 16
NW = NC * NS
NB = 4


def _ceil_to(x, m):
    return ((x + m - 1) // m) * m


@functools.lru_cache(maxsize=None)
def _make_feat2(e1, acc1, e2, acc2):
    npcs = (e1 // (NW * K), e2 // (NW * K))
    zs = (acc1 // NS, acc2 // NS)
    accmax = max(acc1, acc2)
    mesh = plsc.VectorSubcoreMesh(core_axis_name="c", subcore_axis_name="s")
    out_type = [
        jax.ShapeDtypeStruct((NC, acc1, D), jnp.float32),
        jax.ShapeDtypeStruct((NC, acc2, D), jnp.float32),
        jax.ShapeDtypeStruct((NC, NS, acc1), jnp.float32),
        jax.ShapeDtypeStruct((NC, NS, acc2), jnp.float32),
    ]
    scratch = [
        pltpu.VMEM_SHARED((accmax, D), jnp.float32),
        pltpu.VMEM((NB, K), jnp.int32),
        pltpu.VMEM((NB, K), jnp.int32),
        pltpu.VMEM((K, D), jnp.float32),
        pltpu.VMEM((K, D), jnp.float32),
        pltpu.VMEM((accmax,), jnp.float32),
        pltpu.SemaphoreType.DMA,
        pltpu.SemaphoreType.DMA,
        pltpu.SemaphoreType.DMA,
        pltpu.SemaphoreType.DMA,
    ]

    @functools.partial(
        pl.kernel, mesh=mesh, out_type=out_type, scratch_types=scratch,
        compiler_params=pltpu.CompilerParams(needs_layout_passes=False))
    def feat2(t1, s1, d1, t2, s2, d2, zf,
              sum1, sum2, deg1, deg2,
              acc, sblk, dblk, rows0, rows1, hist, sg0, sg1, ss0, ss1):
        c = lax.axis_index("c")
        s = lax.axis_index("s")
        wid = s * NC + c
        ones16 = jnp.ones((16,), jnp.float32)
        zero16 = jnp.zeros((16,), jnp.float32)

        for (table, srcr, dstr, npc, z, acc_n, sumo, dego) in (
                (t1, s1, d1, npcs[0], zs[0], acc1, sum1, deg1),
                (t2, s2, d2, npcs[1], zs[1], acc2, sum2, deg2)):
            nzc = z // K
            pltpu.sync_copy(zf, rows0)

            def zbody(i, carry):
                pltpu.sync_copy(rows0, acc.at[pl.ds(s * z + i * K, K)])
                return carry

            lax.fori_loop(0, nzc, zbody, 0)

            def zhist(i, carry):
                hist[pl.ds(i * 16, 16)] = zero16
                return carry

            lax.fori_loop(0, acc_n // 16, zhist, 0)
            plsc.subcore_barrier()
            crow0 = wid * npc

            def body(t, carry):
                crow = crow0 + t * NB
                i0 = pltpu.async_copy(srcr.at[pl.ds(crow, NB)], sblk, sg0)
                i1 = pltpu.async_copy(dstr.at[pl.ds(crow, NB)], dblk, sg1)
                i0.wait()
                i1.wait()
                cprev = None
                for h in range(NB // 2):
                    if cprev is not None:
                        cprev[0].wait()
                    g0 = pltpu.async_copy(table.at[sblk.at[2 * h]],
                                          rows0, sg0)
                    if cprev is not None:
                        cprev[1].wait()
                    g1 = pltpu.async_copy(table.at[sblk.at[2 * h + 1]],
                                          rows1, sg1)
                    for u in (2 * h, 2 * h + 1):
                        for j in range(K // 16):
                            idx16 = dblk[u, pl.ds(j * 16, 16)]
                            plsc.addupdate_scatter(hist, [idx16], ones16)
                    g0.wait()
                    c0 = pltpu.async_copy(rows0, acc.at[dblk.at[2 * h]],
                                          ss0, add=True)
                    g1.wait()
                    c1 = pltpu.async_copy(rows1, acc.at[dblk.at[2 * h + 1]],
                                          ss1, add=True)
                    cprev = (c0, c1)
                cprev[0].wait()
                cprev[1].wait()
                return carry

            lax.fori_loop(0, npc // NB, body, 0)
            plsc.subcore_barrier()
            pltpu.sync_copy(hist.at[pl.ds(0, acc_n)], dego.at[c, s])

            def obody(i, carry):
                r0 = s * z + i * K
                pltpu.sync_copy(acc.at[pl.ds(r0, K)], rows0)
                pltpu.sync_copy(rows0, sumo.at[c, pl.ds(r0, K)])
                return carry

            lax.fori_loop(0, nzc, obody, 0)
            plsc.subcore_barrier()

    return feat2


def _pad_edges(ei, n_dst):
    src, dst = ei[0], ei[1]
    e = src.shape[0]
    epad = _ceil_to(e, NW * K * NB)
    pad = epad - e
    if pad:
        src = jnp.concatenate([src, jnp.zeros((pad,), jnp.int32)])
        dst = jnp.concatenate([dst, jnp.full((pad,), n_dst, jnp.int32)])
    return src.reshape(epad // K, K), dst.reshape(epad // K, K), epad


def _mean_linear(sums, degs, W, b, n, leaky):
    blk = 1000
    nb = n // blk

    def body(s_ref, d_ref, w_ref, b_ref, o_ref):
        ss = s_ref[...]
        dd = d_ref[...]
        sm = ss[0] + ss[1]
        d = jnp.sum(dd, axis=1)[:, None]
        mean = sm / jnp.maximum(d, 1.0)
        h = jnp.dot(mean, w_ref[...], preferred_element_type=jnp.float32)
        h = h + jnp.where(d > 0, b_ref[...], 0.0)
        if leaky:
            h = jnp.where(h >= 0, h, 0.01 * h)
        o_ref[...] = h

    return pl.pallas_call(
        body,
        grid=(nb,),
        in_specs=[
            pl.BlockSpec((NC, blk, D), lambda i: (0, i, 0)),
            pl.BlockSpec((blk, NW), lambda i: (i, 0)),
            pl.BlockSpec((D, D), lambda i: (0, 0)),
            pl.BlockSpec((1, D), lambda i: (0, 0)),
        ],
        out_specs=pl.BlockSpec((blk, D), lambda i: (i, 0)),
        out_shape=jax.ShapeDtypeStruct((n, D), jnp.float32),
    )(sums, degs, W, b.reshape(1, D))


def _final_combine(sw, dw, Ww, bw, st, dt, Wt, bt, n):
    blk = 1000
    nb = n // blk

    def body(sw_ref, dw_ref, ww_ref, bw_ref, st_ref, dt_ref, wt_ref, bt_ref,
             o_ref):
        out = None
        for s_ref, d_ref, w_ref, b_ref in (
                (sw_ref, dw_ref, ww_ref, bw_ref),
                (st_ref, dt_ref, wt_ref, bt_ref)):
            ss = s_ref[...]
            dd = d_ref[...]
            sm = ss[0] + ss[1]
            d = jnp.sum(dd, axis=1)[:, None]
            mean = sm / jnp.maximum(d, 1.0)
            h = jnp.dot(mean, w_ref[...], preferred_element_type=jnp.float32)
            h = h + jnp.where(d > 0, b_ref[...], 0.0)
            out = h if out is None else out + h
        o_ref[...] = out

    mat = pl.BlockSpec((NC, blk, D), lambda i: (0, i, 0))
    deg = pl.BlockSpec((blk, NW), lambda i: (i, 0))
    wsp = pl.BlockSpec((D, D), lambda i: (0, 0))
    bsp = pl.BlockSpec((1, D), lambda i: (0, 0))
    return pl.pallas_call(
        body,
        grid=(nb,),
        in_specs=[mat, deg, wsp, bsp, mat, deg, wsp, bsp],
        out_specs=pl.BlockSpec((blk, D), lambda i: (i, 0)),
        out_shape=jax.ShapeDtypeStruct((n, D), jnp.float32),
    )(sw, dw, Ww, bw.reshape(1, D), st, dt, Wt, bt.reshape(1, D))


def kernel(embeds, params, edges):
    paper = embeds["paper"]
    n_author, n_paper, n_field = 10000, 10000, 5000

    W_wb, b_wb = params["layer0"]["paper,written_by,author"]
    W_ht, b_ht = params["layer0"]["paper,has_topic,field"]
    W_w, b_w = params["layer1"]["author,writes,paper"]
    W_t, b_t = params["layer1"]["field,topic_of,paper"]

    s_wb, d_wb, e_wb = _pad_edges(edges["paper,written_by,author"], n_author)
    s_ht, d_ht, e_ht = _pad_edges(edges["paper,has_topic,field"], n_field)
    s_w, d_w, e_w = _pad_edges(edges["author,writes,paper"], n_paper)
    s_t, d_t, e_t = _pad_edges(edges["field,topic_of,paper"], n_paper)

    acc_a = _ceil_to(n_author + 8, NS * K)
    acc_f = _ceil_to(n_field + 8, NS * K)
    acc_p = _ceil_to(n_paper + 8, NS * K)
    zf = jnp.zeros((K, D), jnp.float32)

    sum_a, sum_f, deg_a, deg_f = _make_feat2(e_wb, acc_a, e_ht, acc_f)(
        paper, s_wb, d_wb, paper, s_ht, d_ht, zf)
    h_a = _mean_linear(sum_a[:, :n_author],
                       deg_a.reshape(NW, -1).T[:n_author],
                       W_wb, b_wb, n_author, leaky=True)
    h_f = _mean_linear(sum_f[:, :n_field],
                       deg_f.reshape(NW, -1).T[:n_field],
                       W_ht, b_ht, n_field, leaky=True)

    sum_w, sum_t, deg_w, deg_t = _make_feat2(e_w, acc_p, e_t, acc_p)(
        h_a, s_w, d_w, h_f, s_t, d_t, zf)
    return _final_combine(sum_w[:, :n_paper],
                          deg_w.reshape(NW, -1).T[:n_paper], W_w, b_w,
                          sum_t[:, :n_paper],
                          deg_t.reshape(NW, -1).T[:n_paper], W_t, b_t,
                          n_paper)


# speedup vs baseline: 3.2645x; 1.0860x over previous
import functools

import jax
import jax.numpy as jnp
from jax import lax
from jax.experimental import pallas as pl
from jax.experimental.pallas import tpu as pltpu
from jax.experimental.pallas import tpu_sc as plsc

D = 128
K = 128
NC = 2
NS = 16
NW = NC * NS
NB = 4


def _ceil_to(x, m):
    return ((x + m - 1) // m) * m


@functools.lru_cache(maxsize=None)
def _make_feat2(e1, acc1, e2, acc2, c0_frac_num=1, c0_frac_den=2):
    def _split(e):
        tot = e // (NS * K)
        q0 = _ceil_to((tot * c0_frac_num) // c0_frac_den, NB)
        return q0, tot - q0
    qs = (_split(e1), _split(e2))
    zs = (acc1 // NS, acc2 // NS)
    accmax = max(acc1, acc2)
    mesh = plsc.VectorSubcoreMesh(core_axis_name="c", subcore_axis_name="s")
    out_type = [
        jax.ShapeDtypeStruct((NC, acc1, D), jnp.float32),
        jax.ShapeDtypeStruct((NC, acc2, D), jnp.float32),
        jax.ShapeDtypeStruct((NC, NS, acc1), jnp.float32),
        jax.ShapeDtypeStruct((NC, NS, acc2), jnp.float32),
    ]
    scratch = [
        pltpu.VMEM_SHARED((accmax, D), jnp.float32),
        pltpu.VMEM((NB, K), jnp.int32),
        pltpu.VMEM((NB, K), jnp.int32),
        pltpu.VMEM((K, D), jnp.float32),
        pltpu.VMEM((K, D), jnp.float32),
        pltpu.VMEM((accmax,), jnp.float32),
        pltpu.SemaphoreType.DMA,
        pltpu.SemaphoreType.DMA,
        pltpu.SemaphoreType.DMA,
        pltpu.SemaphoreType.DMA,
    ]

    @functools.partial(
        pl.kernel, mesh=mesh, out_type=out_type, scratch_types=scratch,
        compiler_params=pltpu.CompilerParams(needs_layout_passes=False))
    def feat2(t1, s1, d1, t2, s2, d2, zf,
              sum1, sum2, deg1, deg2,
              acc, sblk, dblk, rows0, rows1, hist, sg0, sg1, ss0, ss1):
        c = lax.axis_index("c")
        s = lax.axis_index("s")
        wid = s * NC + c
        ones16 = jnp.ones((16,), jnp.float32)
        zero16 = jnp.zeros((16,), jnp.float32)

        for (table, srcr, dstr, (q0, q1), z, acc_n, sumo, dego) in (
                (t1, s1, d1, qs[0], zs[0], acc1, sum1, deg1),
                (t2, s2, d2, qs[1], zs[1], acc2, sum2, deg2)):
            nzc = z // K
            pltpu.sync_copy(zf, rows0)

            def zbody(i, carry):
                pltpu.sync_copy(rows0, acc.at[pl.ds(s * z + i * K, K)])
                return carry

            lax.fori_loop(0, nzc, zbody, 0)

            def zhist(i, carry):
                hist[pl.ds(i * 16, 16)] = zero16
                return carry

            lax.fori_loop(0, acc_n // 16, zhist, 0)
            plsc.subcore_barrier()
            crow0 = jnp.where(c == 0, s * q0, NS * q0 + s * q1)
            nblk = jnp.where(c == 0, q0 // NB, q1 // NB)

            def body(t, carry):
                crow = crow0 + t * NB
                i0 = pltpu.async_copy(srcr.at[pl.ds(crow, NB)], sblk, sg0)
                i1 = pltpu.async_copy(dstr.at[pl.ds(crow, NB)], dblk, sg1)
                i0.wait()
                i1.wait()
                cprev = None
                for h in range(NB // 2):
                    if cprev is not None:
                        cprev[0].wait()
                    g0 = pltpu.async_copy(table.at[sblk.at[2 * h]],
                                          rows0, sg0)
                    if cprev is not None:
                        cprev[1].wait()
                    g1 = pltpu.async_copy(table.at[sblk.at[2 * h + 1]],
                                          rows1, sg1)
                    for u in (2 * h, 2 * h + 1):
                        for j in range(K // 16):
                            idx16 = dblk[u, pl.ds(j * 16, 16)]
                            plsc.addupdate_scatter(hist, [idx16], ones16)
                    g0.wait()
                    c0 = pltpu.async_copy(rows0, acc.at[dblk.at[2 * h]],
                                          ss0, add=True)
                    g1.wait()
                    c1 = pltpu.async_copy(rows1, acc.at[dblk.at[2 * h + 1]],
                                          ss1, add=True)
                    cprev = (c0, c1)
                cprev[0].wait()
                cprev[1].wait()
                return carry

            lax.fori_loop(0, nblk, body, 0)
            plsc.subcore_barrier()
            pltpu.sync_copy(hist.at[pl.ds(0, acc_n)], dego.at[c, s])

            def obody(i, carry):
                r0 = s * z + i * K
                pltpu.sync_copy(acc.at[pl.ds(r0, K)], rows0)
                pltpu.sync_copy(rows0, sumo.at[c, pl.ds(r0, K)])
                return carry

            lax.fori_loop(0, nzc, obody, 0)
            plsc.subcore_barrier()

    return feat2


def _pad_edges(ei, n_dst):
    src, dst = ei[0], ei[1]
    e = src.shape[0]
    epad = _ceil_to(e, NW * K * NB)
    pad = epad - e
    if pad:
        src = jnp.concatenate([src, jnp.zeros((pad,), jnp.int32)])
        dst = jnp.concatenate([dst, jnp.full((pad,), n_dst, jnp.int32)])
    return src.reshape(epad // K, K), dst.reshape(epad // K, K), epad


def _mean_linear(sums, degs, W, b, n, leaky):
    blk = 1000
    nb = n // blk

    def body(s_ref, d_ref, w_ref, b_ref, o_ref):
        ss = s_ref[...]
        dd = d_ref[...]
        sm = ss[0] + ss[1]
        d = jnp.sum(dd, axis=1)[:, None]
        mean = sm / jnp.maximum(d, 1.0)
        h = jnp.dot(mean, w_ref[...], preferred_element_type=jnp.float32)
        h = h + jnp.where(d > 0, b_ref[...], 0.0)
        if leaky:
            h = jnp.where(h >= 0, h, 0.01 * h)
        o_ref[...] = h

    return pl.pallas_call(
        body,
        grid=(nb,),
        in_specs=[
            pl.BlockSpec((NC, blk, D), lambda i: (0, i, 0)),
            pl.BlockSpec((blk, NW), lambda i: (i, 0)),
            pl.BlockSpec((D, D), lambda i: (0, 0)),
            pl.BlockSpec((1, D), lambda i: (0, 0)),
        ],
        out_specs=pl.BlockSpec((blk, D), lambda i: (i, 0)),
        out_shape=jax.ShapeDtypeStruct((n, D), jnp.float32),
    )(sums, degs, W, b.reshape(1, D))


def _final_combine(sw, dw, Ww, bw, st, dt, Wt, bt, n):
    blk = 1000
    nb = n // blk

    def body(sw_ref, dw_ref, ww_ref, bw_ref, st_ref, dt_ref, wt_ref, bt_ref,
             o_ref):
        out = None
        for s_ref, d_ref, w_ref, b_ref in (
                (sw_ref, dw_ref, ww_ref, bw_ref),
                (st_ref, dt_ref, wt_ref, bt_ref)):
            ss = s_ref[...]
            dd = d_ref[...]
            sm = ss[0] + ss[1]
            d = jnp.sum(dd, axis=1)[:, None]
            mean = sm / jnp.maximum(d, 1.0)
            h = jnp.dot(mean, w_ref[...], preferred_element_type=jnp.float32)
            h = h + jnp.where(d > 0, b_ref[...], 0.0)
            out = h if out is None else out + h
        o_ref[...] = out

    mat = pl.BlockSpec((NC, blk, D), lambda i: (0, i, 0))
    deg = pl.BlockSpec((blk, NW), lambda i: (i, 0))
    wsp = pl.BlockSpec((D, D), lambda i: (0, 0))
    bsp = pl.BlockSpec((1, D), lambda i: (0, 0))
    return pl.pallas_call(
        body,
        grid=(nb,),
        in_specs=[mat, deg, wsp, bsp, mat, deg, wsp, bsp],
        out_specs=pl.BlockSpec((blk, D), lambda i: (i, 0)),
        out_shape=jax.ShapeDtypeStruct((n, D), jnp.float32),
    )(sw, dw, Ww, bw.reshape(1, D), st, dt, Wt, bt.reshape(1, D))


def kernel(embeds, params, edges):
    paper = embeds["paper"]
    n_author, n_paper, n_field = 10000, 10000, 5000

    W_wb, b_wb = params["layer0"]["paper,written_by,author"]
    W_ht, b_ht = params["layer0"]["paper,has_topic,field"]
    W_w, b_w = params["layer1"]["author,writes,paper"]
    W_t, b_t = params["layer1"]["field,topic_of,paper"]

    s_wb, d_wb, e_wb = _pad_edges(edges["paper,written_by,author"], n_author)
    s_ht, d_ht, e_ht = _pad_edges(edges["paper,has_topic,field"], n_field)
    s_w, d_w, e_w = _pad_edges(edges["author,writes,paper"], n_paper)
    s_t, d_t, e_t = _pad_edges(edges["field,topic_of,paper"], n_paper)

    acc_a = _ceil_to(n_author + 8, NS * K)
    acc_f = _ceil_to(n_field + 8, NS * K)
    acc_p = _ceil_to(n_paper + 8, NS * K)
    zf = jnp.zeros((K, D), jnp.float32)

    sum_a, sum_f, deg_a, deg_f = _make_feat2(e_wb, acc_a, e_ht, acc_f, 7, 10)(
        paper, s_wb, d_wb, paper, s_ht, d_ht, zf)
    h_a = _mean_linear(sum_a[:, :n_author],
                       deg_a.reshape(NW, -1).T[:n_author],
                       W_wb, b_wb, n_author, leaky=True)
    h_f = _mean_linear(sum_f[:, :n_field],
                       deg_f.reshape(NW, -1).T[:n_field],
                       W_ht, b_ht, n_field, leaky=True)

    sum_w, sum_t, deg_w, deg_t = _make_feat2(e_w, acc_p, e_t, acc_p, 7, 10)(
        h_a, s_w, d_w, h_f, s_t, d_t, zf)
    return _final_combine(sum_w[:, :n_paper],
                          deg_w.reshape(NW, -1).T[:n_paper], W_w, b_w,
                          sum_t[:, :n_paper],
                          deg_t.reshape(NW, -1).T[:n_paper], W_t, b_t,
                          n_paper)
